# baseline (speedup 1.0000x reference)
import functools

import jax
import jax.numpy as jnp
from jax import lax
from jax.experimental import pallas as pl
from jax.experimental.pallas import tpu as pltpu
from jax.experimental.pallas import tpu_sc as plsc

NC = 2
NS = 16
NW = NC * NS
L = 16

D = 128
K = 256
KH = 128
NBUF = 2


def _sc_lookup(idx2d, trainable, fixed):
    n_rows_total, kh = idx2d.shape
    assert kh == KH and n_rows_total % (2 * NW) == 0
    rows_w = n_rows_total // NW
    n_chunks = rows_w // 2
    n_chunks_total = n_chunks * NW
    assert n_chunks % NBUF == 0
    b_total = n_chunks_total * K
    train_n = trainable.shape[0]
    mesh = plsc.VectorSubcoreMesh(core_axis_name="c", subcore_axis_name="s")

    slot_scratch = []
    for _ in range(NBUF):
        slot_scratch += [
            pltpu.VMEM((KH,), jnp.int32),
            pltpu.VMEM((KH,), jnp.int32),
            pltpu.VMEM((K, D), jnp.float32),
            pltpu.VMEM((K + L,), jnp.int32),
            pltpu.VMEM((K + L,), jnp.int32),
            pltpu.SemaphoreType.DMA,
            pltpu.SemaphoreType.DMA,
        ]

    @functools.partial(
        pl.kernel,
        out_type=jax.ShapeDtypeStruct((b_total, D), jnp.float32),
        mesh=mesh,
        scratch_types=[pltpu.VMEM((2 * n_chunks, KH), jnp.int32)] + slot_scratch
        + [pltpu.SemaphoreType.DMA],
        compiler_params=pltpu.CompilerParams(needs_layout_passes=False),
    )
    def k_fn(idx_hbm, train_hbm, fixed_hbm, out_hbm, idxall, *rest):
        slots = [tuple(rest[i * 7:(i + 1) * 7]) for i in range(NBUF)]
        psem = rest[NBUF * 7]
        wid = lax.axis_index("s") * NC + lax.axis_index("c")
        row0 = wid * (n_chunks * K)
        pltpu.sync_copy(idx_hbm.at[pl.ds(wid * rows_w, rows_w)], idxall)

        def front(c, s):
            fidxA, fidxB, buf, jl, tl, gsem, _ = slots[s]

            def make_grp(fidx_h, goff, half):
                def grp(g, off):
                    v = idxall[2 * c + half, pl.ds((g - goff) * L, L)]
                    is_tr = v < train_n
                    jvec = lax.iota(jnp.int32, L) + g * L
                    spread = (row0 + c * K + jvec) & 0xFFFF
                    fidx_h[pl.ds((g - goff) * L, L)] = jnp.where(
                        is_tr, spread, v - train_n)
                    pfx = plsc.cumsum(is_tr.astype(jnp.int32))
                    lanes = off + pfx - 1
                    plsc.store_scatter(jl, [lanes], jvec, mask=is_tr)
                    plsc.store_scatter(tl, [lanes], v, mask=is_tr)
                    return off + pfx[L - 1]

                return grp

            off1 = lax.fori_loop(0, KH // L, make_grp(fidxA, 0, 0),
                                 jnp.int32(0))
            pltpu.async_copy(fixed_hbm.at[fidxA], buf.at[pl.ds(0, KH)], gsem)
            n_tr = lax.fori_loop(KH // L, K // L, make_grp(fidxB, KH // L, 1),
                                 off1)
            pltpu.async_copy(fixed_hbm.at[fidxB], buf.at[pl.ds(KH, KH)], gsem)
            return n_tr

        def finish(s, base, n_tr):
            fidxA, fidxB, buf, jl, tl, gsem, wsem = slots[s]
            pltpu.make_async_copy(fixed_hbm.at[fidxA], buf.at[pl.ds(0, KH)],
                                  gsem).wait()
            pltpu.make_async_copy(fixed_hbm.at[fidxB], buf.at[pl.ds(KH, KH)],
                                  gsem).wait()

            def patch_issue(i, _):
                j = jl[pl.ds(i, L)][0]
                t = tl[pl.ds(i, L)][0]
                pltpu.async_copy(train_hbm.at[t], buf.at[j], psem)
                return 0

            def patch_drain(i, _):
                pltpu.make_async_copy(train_hbm.at[0], buf.at[0], psem).wait()
                return 0

            lax.fori_loop(0, n_tr, patch_issue, 0)
            lax.fori_loop(0, n_tr, patch_drain, 0)
            pltpu.async_copy(buf, out_hbm.at[pl.ds(base, K)], wsem)

        def step(st, ntrs):
            ntrs = list(ntrs)
            for s in range(NBUF):
                c = st * NBUF + s
                buf_s, wsem_s = slots[s][2], slots[s][6]

                @pl.when(c >= NBUF)
                def _():
                    pltpu.make_async_copy(buf_s, out_hbm.at[pl.ds(row0, K)],
                                          wsem_s).wait()

                ntrs[s] = lax.cond(c < n_chunks, lambda c=c, s=s: front(c, s),
                                   lambda: jnp.int32(0))

                sf = (s + 1) % NBUF
                cf = c - (NBUF - 1)

                @pl.when((cf >= 0) & (cf < n_chunks))
                def _():
                    finish(sf, row0 + cf * K, ntrs[sf])

            return tuple(ntrs)

        lax.fori_loop(0, n_chunks // NBUF + 1, step,
                      (jnp.int32(0),) * NBUF)

    return k_fn(idx2d, trainable, fixed)


def kernel(indices, trainable_embedding, fixed_embedding):
    b, t = indices.shape
    idx2d = indices.reshape(-1, KH).astype(jnp.int32)
    out = _sc_lookup(idx2d, trainable_embedding, fixed_embedding)
    return out.reshape(b, t, D)


# speedup vs baseline: 1.0048x; 1.0048x over previous
import functools

import jax
import jax.numpy as jnp
from jax import lax
from jax.experimental import pallas as pl
from jax.experimental.pallas import tpu as pltpu
from jax.experimental.pallas import tpu_sc as plsc

NC = 2
NS = 16
NW = NC * NS
L = 16

D = 128
K = 256
KH = 128
NBUF = 2


def _sc_lookup(idx2d, trainable, fixed):
    n_rows_total, kh = idx2d.shape
    assert kh == KH and n_rows_total % (2 * NW) == 0
    rows_w = n_rows_total // NW
    n_chunks = rows_w // 2
    n_chunks_total = n_chunks * NW
    assert n_chunks % NBUF == 0
    b_total = n_chunks_total * K
    train_n = trainable.shape[0]
    mesh = plsc.VectorSubcoreMesh(core_axis_name="c", subcore_axis_name="s")

    slot_scratch = []
    for _ in range(NBUF):
        slot_scratch += [
            pltpu.VMEM((KH,), jnp.int32),
            pltpu.VMEM((KH,), jnp.int32),
            pltpu.VMEM((K, D), jnp.float32),
            pltpu.VMEM((K + L,), jnp.int32),
            pltpu.VMEM((K + L,), jnp.int32),
            pltpu.SemaphoreType.DMA,
            pltpu.SemaphoreType.DMA,
        ]

    @functools.partial(
        pl.kernel,
        out_type=jax.ShapeDtypeStruct((b_total, D), jnp.float32),
        mesh=mesh,
        scratch_types=[pltpu.VMEM((2 * n_chunks, KH), jnp.int32)] + slot_scratch
        + [pltpu.SemaphoreType.DMA],
        compiler_params=pltpu.CompilerParams(needs_layout_passes=False),
    )
    def k_fn(idx_hbm, train_hbm, fixed_hbm, out_hbm, idxall, *rest):
        slots = [tuple(rest[i * 7:(i + 1) * 7]) for i in range(NBUF)]
        psem = rest[NBUF * 7]
        wid = lax.axis_index("s") * NC + lax.axis_index("c")
        row0 = wid * (n_chunks * K)
        pltpu.sync_copy(idx_hbm.at[pl.ds(wid * rows_w, rows_w)], idxall)

        def front(c, s):
            fidxA, fidxB, buf, jl, tl, gsem, _ = slots[s]

            def make_grp(fidx_h, goff, half):
                def grp(g, off):
                    v = idxall[2 * c + half, pl.ds((g - goff) * L, L)]
                    is_tr = v < train_n
                    jvec = lax.iota(jnp.int32, L) + g * L
                    spread = (row0 + c * K + jvec) & 0xFFFF
                    fidx_h[pl.ds((g - goff) * L, L)] = jnp.where(
                        is_tr, spread, v - train_n)
                    pfx = plsc.cumsum(is_tr.astype(jnp.int32))
                    lanes = off + pfx - 1
                    plsc.store_scatter(jl, [lanes], jvec, mask=is_tr)
                    plsc.store_scatter(tl, [lanes], v, mask=is_tr)
                    return off + pfx[L - 1]

                return grp

            off1 = lax.fori_loop(0, KH // L, make_grp(fidxA, 0, 0),
                                 jnp.int32(0))
            n_tr = lax.fori_loop(KH // L, K // L, make_grp(fidxB, KH // L, 1),
                                 off1)
            pltpu.async_copy(fixed_hbm.at[fidxA], buf.at[pl.ds(0, KH)], gsem)
            pltpu.async_copy(fixed_hbm.at[fidxB], buf.at[pl.ds(KH, KH)], gsem)
            return n_tr

        def finish(s, base, n_tr):
            fidxA, fidxB, buf, jl, tl, gsem, wsem = slots[s]
            pltpu.make_async_copy(fixed_hbm.at[fidxA], buf.at[pl.ds(0, KH)],
                                  gsem).wait()
            pltpu.make_async_copy(fixed_hbm.at[fidxB], buf.at[pl.ds(KH, KH)],
                                  gsem).wait()

            def patch_issue(i, _):
                j = jl[pl.ds(i, L)][0]
                t = tl[pl.ds(i, L)][0]
                pltpu.async_copy(train_hbm.at[t], buf.at[j], psem)
                return 0

            def patch_drain(i, _):
                pltpu.make_async_copy(train_hbm.at[0], buf.at[0], psem).wait()
                return 0

            lax.fori_loop(0, n_tr, patch_issue, 0)
            lax.fori_loop(0, n_tr, patch_drain, 0)
            pltpu.async_copy(buf, out_hbm.at[pl.ds(base, K)], wsem)

        def step(st, ntrs):
            ntrs = list(ntrs)
            for s in range(NBUF):
                c = st * NBUF + s
                buf_s, wsem_s = slots[s][2], slots[s][6]

                @pl.when(c >= NBUF)
                def _():
                    pltpu.make_async_copy(buf_s, out_hbm.at[pl.ds(row0, K)],
                                          wsem_s).wait()

                ntrs[s] = lax.cond(c < n_chunks, lambda c=c, s=s: front(c, s),
                                   lambda: jnp.int32(0))

                sf = (s + 1) % NBUF
                cf = c - (NBUF - 1)

                @pl.when((cf >= 0) & (cf < n_chunks))
                def _():
                    finish(sf, row0 + cf * K, ntrs[sf])

            return tuple(ntrs)

        lax.fori_loop(0, n_chunks // NBUF + 1, step,
                      (jnp.int32(0),) * NBUF)

    return k_fn(idx2d, trainable, fixed)


def kernel(indices, trainable_embedding, fixed_embedding):
    b, t = indices.shape
    idx2d = indices.reshape(-1, KH).astype(jnp.int32)
    out = _sc_lookup(idx2d, trainable_embedding, fixed_embedding)
    return out.reshape(b, t, D)
